# Initial kernel scaffold; baseline (speedup 1.0000x reference)
import functools

import jax
import jax.numpy as jnp
from jax import lax
from jax.experimental import pallas as pl
from jax.experimental.pallas import tpu as pltpu
from jax.experimental.pallas import tpu_sc as plsc

N = 10000
E = 320000
D = 128
KATTR = 5

NC = 2
NS = 16
NW = NC * NS
EPW = E // NW
CHUNK = 80
NCH = EPW // CHUNK
RPT = N // NS
ZR = 125

_mesh = plsc.VectorSubcoreMesh(core_axis_name="c", subcore_axis_name="s")


@functools.partial(
    pl.kernel,
    out_type=jax.ShapeDtypeStruct((2 * N,), jnp.float32),
    mesh=_mesh,
    scratch_types=[
        pltpu.VMEM((CHUNK,), jnp.int32),
        pltpu.VMEM((CHUNK,), jnp.float32),
        pltpu.VMEM((1000,), jnp.float32),
        pltpu.VMEM_SHARED((N,), jnp.float32),
    ],
)
def _sc_degree(dst, ones, zeros1k, out, didx, onesv, zbuf, accd):
    c = lax.axis_index("c")
    s = lax.axis_index("s")
    wid = c * NS + s
    pltpu.sync_copy(ones, onesv)

    @pl.when(s < 10)
    def _zero():
        pltpu.sync_copy(zeros1k, zbuf)
        pltpu.sync_copy(zbuf, accd.at[pl.ds(s * 1000, 1000)])

    plsc.subcore_barrier()
    base = wid * EPW

    def body(i, carry):
        off = base + i * CHUNK
        pltpu.sync_copy(dst.at[pl.ds(off, CHUNK)], didx)
        pltpu.sync_copy(onesv, accd.at[didx], add=True)
        return carry

    lax.fori_loop(0, NCH, body, 0)
    plsc.subcore_barrier()

    @pl.when(s < 10)
    def _copy_out():
        pltpu.sync_copy(accd.at[pl.ds(s * 1000, 1000)], zbuf)
        pltpu.sync_copy(zbuf, out.at[pl.ds(c * N + s * 1000, 1000)])


@functools.partial(
    pl.kernel,
    out_type=jax.ShapeDtypeStruct((2 * N, D), jnp.float32),
    mesh=_mesh,
    scratch_types=[
        pltpu.VMEM((CHUNK,), jnp.int32),
        pltpu.VMEM((CHUNK,), jnp.int32),
        pltpu.VMEM((CHUNK, D), jnp.float32),
        pltpu.VMEM((ZR, D), jnp.float32),
        pltpu.VMEM_SHARED((N, D), jnp.float32),
        pltpu.SemaphoreType.DMA,
    ],
)
def _sc_aggregate(table, src, dst, zeros, out, sidx, didx, rows, zbuf, acc,
                  gsem):
    c = lax.axis_index("c")
    s = lax.axis_index("s")
    wid = c * NS + s
    pltpu.sync_copy(zeros, zbuf)
    for j in range(RPT // ZR):
        pltpu.sync_copy(zbuf, acc.at[pl.ds(s * RPT + j * ZR, ZR)])
    plsc.subcore_barrier()
    base = wid * EPW

    def body(i, carry):
        off = base + i * CHUNK
        pltpu.sync_copy(src.at[pl.ds(off, CHUNK)], sidx)
        pltpu.sync_copy(dst.at[pl.ds(off, CHUNK)], didx)
        pltpu.async_copy(table.at[sidx], rows, gsem).wait()
        pltpu.sync_copy(rows, acc.at[didx], add=True)
        return carry

    lax.fori_loop(0, NCH, body, 0)
    plsc.subcore_barrier()
    for j in range(RPT // ZR):
        r0 = s * RPT + j * ZR
        pltpu.sync_copy(acc.at[pl.ds(r0, ZR)], zbuf)
        pltpu.sync_copy(zbuf, out.at[pl.ds(c * N + r0, ZR)])


BLK = 400
GRID = N // BLK

_row = pl.BlockSpec((BLK, D), lambda i: (i, 0))
_col = pl.BlockSpec((BLK, 1), lambda i: (i, 0))
_w = pl.BlockSpec((D, D), lambda i: (0, 0))
_b = pl.BlockSpec((1, D), lambda i: (0, 0))
_wcol = pl.BlockSpec((D, 1), lambda i: (0, 0))
_scal = pl.BlockSpec((1, 1), lambda i: (0, 0))


def _dot(a, b):
    return jnp.dot(a, b, preferred_element_type=jnp.float32)


def _tc1_body(x, d0, d1, ep, w_pre, bp, wg1, wd0, bd0, wd1, bd1, wd2, bd2,
              wd3, bd3, wfb, t1_o, dinv_o, yd_o):
    dinv = lax.rsqrt(d0[...] + d1[...] + 1.0)
    dinv_o[...] = dinv
    h0 = _dot(x[...], w_pre[...]) + bp[...]
    t1_o[...] = _dot(h0, wg1[...]) * dinv
    e = jnp.maximum(_dot(ep[...], wd0[...]) + bd0[...], 0.0)
    e = jnp.maximum(_dot(e, wd1[...]) + bd1[...], 0.0)
    e = jnp.maximum(_dot(e, wd2[...]) + bd2[...], 0.0)
    dist = _dot(e, wd3[...]) + bd3[...]
    yd_o[...] = _dot(dist, wfb[...])


_tc1 = pl.pallas_call(
    _tc1_body,
    grid=(GRID,),
    in_specs=[_row, _col, _col, _row, _w, _b, _w, _w, _b, _w, _b, _w, _b,
              _w, _b, _wcol],
    out_specs=[_row, _col, _col],
    out_shape=[
        jax.ShapeDtypeStruct((N, D), jnp.float32),
        jax.ShapeDtypeStruct((N, 1), jnp.float32),
        jax.ShapeDtypeStruct((N, 1), jnp.float32),
    ],
)


def _tc2_body(pa, pb, t1, dinv, bg, wg2, t2_o):
    h = jnp.maximum((pa[...] + pb[...] + t1[...]) * dinv[...] + bg[...], 0.0)
    t2_o[...] = _dot(h, wg2[...]) * dinv[...]


_tc2 = pl.pallas_call(
    _tc2_body,
    grid=(GRID,),
    in_specs=[_row, _row, _row, _col, _b, _w],
    out_specs=_row,
    out_shape=jax.ShapeDtypeStruct((N, D), jnp.float32),
)


def _tc3_body(pa, pb, t2, dinv, bg, w_post, b_post, wfa, yd, bfin, y_o):
    h = jnp.maximum((pa[...] + pb[...] + t2[...]) * dinv[...] + bg[...], 0.0)
    feat = _dot(h, w_post[...]) + b_post[...]
    z = _dot(feat, wfa[...]) + yd[...] + bfin[...]
    y_o[...] = jax.nn.sigmoid(z)


_tc3 = pl.pallas_call(
    _tc3_body,
    grid=(GRID,),
    in_specs=[_row, _row, _row, _col, _b, _w, _b, _wcol, _col, _scal],
    out_specs=_col,
    out_shape=jax.ShapeDtypeStruct((N, 1), jnp.float32),
)


def kernel(x, edge_index, edge_attr, W_pre, b_pre, Wg1, bg1, Wg2, bg2,
           W_post, b_post, Wd0, bd0, Wd1, bd1, Wd2, bd2, Wd3, bd3,
           W_fin, b_fin):
    src = edge_index[0]
    dst = edge_index[1]
    ones_c = jnp.ones((CHUNK,), jnp.float32)
    zeros1k = jnp.zeros((1000,), jnp.float32)
    zeros2d = jnp.zeros((ZR, D), jnp.float32)

    degs = _sc_degree(dst, ones_c, zeros1k)
    d0 = degs[:N].reshape(N, 1)
    d1 = degs[N:].reshape(N, 1)

    ep = jnp.pad(edge_attr, ((0, 0), (0, D - KATTR)))
    wd0p = jnp.pad(Wd0, ((0, D - KATTR), (0, 0)))

    t1, dinv, yd = _tc1(
        x, d0, d1, ep, W_pre, b_pre.reshape(1, D), Wg1, wd0p,
        bd0.reshape(1, D), Wd1, bd1.reshape(1, D), Wd2, bd2.reshape(1, D),
        Wd3, bd3.reshape(1, D), W_fin[D:])

    p1 = _sc_aggregate(t1, src, dst, zeros2d)
    t2 = _tc2(p1[:N], p1[N:], t1, dinv, bg1.reshape(1, D), Wg2)

    p2 = _sc_aggregate(t2, src, dst, zeros2d)
    y = _tc3(p2[:N], p2[N:], t2, dinv, bg2.reshape(1, D), W_post,
             b_post.reshape(1, D), W_fin[:D], yd, b_fin.reshape(1, 1))
    return y.reshape(N)


# speedup vs baseline: 12.4001x; 12.4001x over previous
import functools

import jax
import jax.numpy as jnp
from jax import lax
from jax.experimental import pallas as pl
from jax.experimental.pallas import tpu as pltpu
from jax.experimental.pallas import tpu_sc as plsc

N = 10000
E = 320000
D = 128
KATTR = 5

NC = 2
NS = 16
NW = NC * NS
EPW = E // NW
CHUNK = 80
NCH = EPW // CHUNK
ZR = 80
NZCH = N // ZR

_mesh = plsc.VectorSubcoreMesh(core_axis_name="c", subcore_axis_name="s")


@functools.partial(
    pl.kernel,
    out_type=jax.ShapeDtypeStruct((2 * N,), jnp.float32),
    mesh=_mesh,
    scratch_types=[
        pltpu.VMEM((CHUNK,), jnp.int32),
        pltpu.VMEM((CHUNK,), jnp.float32),
        pltpu.VMEM((1000,), jnp.float32),
        pltpu.VMEM_SHARED((N,), jnp.float32),
    ],
)
def _sc_degree(dst, ones, zeros1k, out, didx, onesv, zbuf, accd):
    c = lax.axis_index("c")
    s = lax.axis_index("s")
    wid = c * NS + s
    pltpu.sync_copy(ones, onesv)

    @pl.when(s < 10)
    def _zero():
        pltpu.sync_copy(zeros1k, zbuf)
        pltpu.sync_copy(zbuf, accd.at[pl.ds(s * 1000, 1000)])

    plsc.subcore_barrier()
    base = wid * EPW

    def body(i, carry):
        off = base + i * CHUNK
        pltpu.sync_copy(dst.at[pl.ds(off, CHUNK)], didx)
        pltpu.sync_copy(onesv, accd.at[didx], add=True)
        return carry

    lax.fori_loop(0, NCH, body, 0)
    plsc.subcore_barrier()

    @pl.when(s < 10)
    def _copy_out():
        pltpu.sync_copy(accd.at[pl.ds(s * 1000, 1000)], zbuf)
        pltpu.sync_copy(zbuf, out.at[pl.ds(c * N + s * 1000, 1000)])


@functools.partial(
    pl.kernel,
    out_type=jax.ShapeDtypeStruct((2 * N, D), jnp.float32),
    mesh=_mesh,
    scratch_types=[
        pltpu.VMEM((CHUNK,), jnp.int32),
        pltpu.VMEM((CHUNK,), jnp.int32),
        pltpu.VMEM((CHUNK, D), jnp.float32),
        pltpu.VMEM((ZR, D), jnp.float32),
        pltpu.VMEM_SHARED((N, D), jnp.float32),
        pltpu.SemaphoreType.DMA,
    ],
)
def _sc_aggregate(table, src, dst, zeros, out, sidx, didx, rows, zbuf, acc,
                  gsem):
    c = lax.axis_index("c")
    s = lax.axis_index("s")
    wid = c * NS + s
    pltpu.sync_copy(zeros, zbuf)
    for t in range((NZCH + NS - 1) // NS):
        ch = s + NS * t

        @pl.when(ch < NZCH)
        def _zero():
            pltpu.sync_copy(zbuf, acc.at[pl.ds(ch * ZR, ZR)])

    plsc.subcore_barrier()
    base = wid * EPW

    def body(i, carry):
        off = base + i * CHUNK
        pltpu.sync_copy(src.at[pl.ds(off, CHUNK)], sidx)
        pltpu.sync_copy(dst.at[pl.ds(off, CHUNK)], didx)
        pltpu.async_copy(table.at[sidx], rows, gsem).wait()
        pltpu.sync_copy(rows, acc.at[didx], add=True)
        return carry

    lax.fori_loop(0, NCH, body, 0)
    plsc.subcore_barrier()
    for t in range((NZCH + NS - 1) // NS):
        ch = s + NS * t

        @pl.when(ch < NZCH)
        def _copy_out():
            pltpu.sync_copy(acc.at[pl.ds(ch * ZR, ZR)], zbuf)
            pltpu.sync_copy(zbuf, out.at[pl.ds(c * N + ch * ZR, ZR)])


BLK = 400
GRID = N // BLK

_row = pl.BlockSpec((BLK, D), lambda i: (i, 0))
_col = pl.BlockSpec((BLK, 1), lambda i: (i, 0))
_w = pl.BlockSpec((D, D), lambda i: (0, 0))
_b = pl.BlockSpec((1, D), lambda i: (0, 0))
_wcol = pl.BlockSpec((D, 1), lambda i: (0, 0))
_scal = pl.BlockSpec((1, 1), lambda i: (0, 0))


def _dot(a, b):
    return jnp.dot(a, b, preferred_element_type=jnp.float32)


def _tc1_body(x, d0, d1, ep, w_pre, bp, wg1, wd0, bd0, wd1, bd1, wd2, bd2,
              wd3, bd3, wfb, t1_o, dinv_o, yd_o):
    dinv = lax.rsqrt(d0[...] + d1[...] + 1.0)
    dinv_o[...] = dinv
    h0 = _dot(x[...], w_pre[...]) + bp[...]
    t1_o[...] = _dot(h0, wg1[...]) * dinv
    e = jnp.maximum(_dot(ep[...], wd0[...]) + bd0[...], 0.0)
    e = jnp.maximum(_dot(e, wd1[...]) + bd1[...], 0.0)
    e = jnp.maximum(_dot(e, wd2[...]) + bd2[...], 0.0)
    dist = _dot(e, wd3[...]) + bd3[...]
    yd_o[...] = _dot(dist, wfb[...])


_tc1 = pl.pallas_call(
    _tc1_body,
    grid=(GRID,),
    in_specs=[_row, _col, _col, _row, _w, _b, _w, _w, _b, _w, _b, _w, _b,
              _w, _b, _wcol],
    out_specs=[_row, _col, _col],
    out_shape=[
        jax.ShapeDtypeStruct((N, D), jnp.float32),
        jax.ShapeDtypeStruct((N, 1), jnp.float32),
        jax.ShapeDtypeStruct((N, 1), jnp.float32),
    ],
)


def _tc2_body(pa, pb, t1, dinv, bg, wg2, t2_o):
    h = jnp.maximum((pa[...] + pb[...] + t1[...]) * dinv[...] + bg[...], 0.0)
    t2_o[...] = _dot(h, wg2[...]) * dinv[...]


_tc2 = pl.pallas_call(
    _tc2_body,
    grid=(GRID,),
    in_specs=[_row, _row, _row, _col, _b, _w],
    out_specs=_row,
    out_shape=jax.ShapeDtypeStruct((N, D), jnp.float32),
)


def _tc3_body(pa, pb, t2, dinv, bg, w_post, b_post, wfa, yd, bfin, y_o):
    h = jnp.maximum((pa[...] + pb[...] + t2[...]) * dinv[...] + bg[...], 0.0)
    feat = _dot(h, w_post[...]) + b_post[...]
    z = _dot(feat, wfa[...]) + yd[...] + bfin[...]
    y_o[...] = jax.nn.sigmoid(z)


_tc3 = pl.pallas_call(
    _tc3_body,
    grid=(GRID,),
    in_specs=[_row, _row, _row, _col, _b, _w, _b, _wcol, _col, _scal],
    out_specs=_col,
    out_shape=jax.ShapeDtypeStruct((N, 1), jnp.float32),
)


def kernel(x, edge_index, edge_attr, W_pre, b_pre, Wg1, bg1, Wg2, bg2,
           W_post, b_post, Wd0, bd0, Wd1, bd1, Wd2, bd2, Wd3, bd3,
           W_fin, b_fin):
    src = edge_index[0]
    dst = edge_index[1]
    ones_c = jnp.ones((CHUNK,), jnp.float32)
    zeros1k = jnp.zeros((1000,), jnp.float32)
    zeros2d = jnp.zeros((ZR, D), jnp.float32)

    degs = _sc_degree(dst, ones_c, zeros1k)
    d0 = degs[:N].reshape(N, 1)
    d1 = degs[N:].reshape(N, 1)

    ep = jnp.pad(edge_attr, ((0, 0), (0, D - KATTR)))
    wd0p = jnp.pad(Wd0, ((0, D - KATTR), (0, 0)))

    t1, dinv, yd = _tc1(
        x, d0, d1, ep, W_pre, b_pre.reshape(1, D), Wg1, wd0p,
        bd0.reshape(1, D), Wd1, bd1.reshape(1, D), Wd2, bd2.reshape(1, D),
        Wd3, bd3.reshape(1, D), W_fin[D:])

    p1 = _sc_aggregate(t1, src, dst, zeros2d)
    t2 = _tc2(p1[:N], p1[N:], t1, dinv, bg1.reshape(1, D), Wg2)

    p2 = _sc_aggregate(t2, src, dst, zeros2d)
    y = _tc3(p2[:N], p2[N:], t2, dinv, bg2.reshape(1, D), W_post,
             b_post.reshape(1, D), W_fin[:D], yd, b_fin.reshape(1, 1))
    return y.reshape(N)


# speedup vs baseline: 25.8434x; 2.0841x over previous
import functools

import jax
import jax.numpy as jnp
from jax import lax
from jax.experimental import pallas as pl
from jax.experimental.pallas import tpu as pltpu
from jax.experimental.pallas import tpu_sc as plsc

N = 10000
E = 320000
D = 128
KATTR = 5

NC = 2
NS = 16
NW = NC * NS
EPW = E // NW
CHUNK = 80
NCH = EPW // CHUNK
ZR = 80
NZCH = N // ZR

_mesh = plsc.VectorSubcoreMesh(core_axis_name="c", subcore_axis_name="s")


@functools.partial(
    pl.kernel,
    out_type=jax.ShapeDtypeStruct((2 * N,), jnp.float32),
    mesh=_mesh,
    scratch_types=[
        pltpu.VMEM((NCH, CHUNK), jnp.int32),
        pltpu.VMEM((CHUNK,), jnp.float32),
        pltpu.VMEM((1000,), jnp.float32),
        pltpu.VMEM_SHARED((N,), jnp.float32),
    ],
)
def _sc_degree(dst3, ones, zeros1k, out, didx, onesv, zbuf, accd):
    c = lax.axis_index("c")
    s = lax.axis_index("s")
    wid = c * NS + s
    pltpu.sync_copy(ones, onesv)
    pltpu.sync_copy(dst3.at[wid], didx)

    @pl.when(s < 10)
    def _zero():
        pltpu.sync_copy(zeros1k, zbuf)
        pltpu.sync_copy(zbuf, accd.at[pl.ds(s * 1000, 1000)])

    plsc.subcore_barrier()

    def body(i, carry):
        pltpu.sync_copy(onesv, accd.at[didx.at[i]], add=True)
        return carry

    lax.fori_loop(0, NCH, body, 0)
    plsc.subcore_barrier()

    @pl.when(s < 10)
    def _copy_out():
        pltpu.sync_copy(accd.at[pl.ds(s * 1000, 1000)], zbuf)
        pltpu.sync_copy(zbuf, out.at[pl.ds(c * N + s * 1000, 1000)])


NBUF = 2


@functools.partial(
    pl.kernel,
    out_type=jax.ShapeDtypeStruct((2 * N, D), jnp.float32),
    mesh=_mesh,
    scratch_types=[
        pltpu.VMEM((EPW,), jnp.int32),
        pltpu.VMEM((NCH, CHUNK), jnp.int32),
        [pltpu.VMEM((CHUNK, D), jnp.float32) for _ in range(NBUF)],
        pltpu.VMEM_SHARED((N, D), jnp.float32),
        [pltpu.SemaphoreType.DMA for _ in range(NBUF)],
    ],
)
def _sc_aggregate(table, src, dst3, zeros, out, sidx, didx, rows, acc, gsem):
    c = lax.axis_index("c")
    s = lax.axis_index("s")
    wid = c * NS + s
    base = wid * EPW
    pltpu.sync_copy(src.at[pl.ds(base, EPW)], sidx)
    pltpu.sync_copy(dst3.at[wid], didx)
    pltpu.sync_copy(zeros, rows[0])
    for t in range((NZCH + NS - 1) // NS):
        ch = s + NS * t

        @pl.when(ch < NZCH)
        def _zero():
            pltpu.sync_copy(rows[0], acc.at[pl.ds(ch * ZR, ZR)])

    plsc.subcore_barrier()
    for b in range(NBUF):
        pltpu.async_copy(table.at[sidx.at[pl.ds(b * CHUNK, CHUNK)]], rows[b],
                         gsem[b])

    def body(k, carry):
        for b in range(NBUF):
            i = k * NBUF + b
            pltpu.make_async_copy(table.at[pl.ds(0, CHUNK)], rows[b],
                                  gsem[b]).wait()
            pltpu.sync_copy(rows[b], acc.at[didx.at[i]], add=True)

            @pl.when(i + NBUF < NCH)
            def _prefetch():
                pltpu.async_copy(
                    table.at[sidx.at[pl.ds((i + NBUF) * CHUNK, CHUNK)]],
                    rows[b], gsem[b])

        return carry

    lax.fori_loop(0, NCH // NBUF, body, 0)
    pltpu.make_async_copy(table.at[pl.ds(0, CHUNK)], rows[0], gsem[0]).wait()
    pltpu.sync_copy(rows[0], acc.at[didx.at[NCH - 1]], add=True)
    plsc.subcore_barrier()
    for t in range((NZCH + NS - 1) // NS):
        ch = s + NS * t

        @pl.when(ch < NZCH)
        def _copy_out():
            pltpu.sync_copy(acc.at[pl.ds(ch * ZR, ZR)], rows[0])
            pltpu.sync_copy(rows[0], out.at[pl.ds(c * N + ch * ZR, ZR)])


BLK = 400
GRID = N // BLK

_row = pl.BlockSpec((BLK, D), lambda i: (i, 0))
_col = pl.BlockSpec((BLK, 1), lambda i: (i, 0))
_w = pl.BlockSpec((D, D), lambda i: (0, 0))
_b = pl.BlockSpec((1, D), lambda i: (0, 0))
_wcol = pl.BlockSpec((D, 1), lambda i: (0, 0))
_scal = pl.BlockSpec((1, 1), lambda i: (0, 0))


def _dot(a, b):
    return jnp.dot(a, b, preferred_element_type=jnp.float32)


def _tc1_body(x, d0, d1, ep, w_pre, bp, wg1, wd0, bd0, wd1, bd1, wd2, bd2,
              wd3, bd3, wfb, t1_o, dinv_o, yd_o):
    dinv = lax.rsqrt(d0[...] + d1[...] + 1.0)
    dinv_o[...] = dinv
    h0 = _dot(x[...], w_pre[...]) + bp[...]
    t1_o[...] = _dot(h0, wg1[...]) * dinv
    e = jnp.maximum(_dot(ep[...], wd0[...]) + bd0[...], 0.0)
    e = jnp.maximum(_dot(e, wd1[...]) + bd1[...], 0.0)
    e = jnp.maximum(_dot(e, wd2[...]) + bd2[...], 0.0)
    dist = _dot(e, wd3[...]) + bd3[...]
    yd_o[...] = _dot(dist, wfb[...])


_tc1 = pl.pallas_call(
    _tc1_body,
    grid=(GRID,),
    in_specs=[_row, _col, _col, _row, _w, _b, _w, _w, _b, _w, _b, _w, _b,
              _w, _b, _wcol],
    out_specs=[_row, _col, _col],
    out_shape=[
        jax.ShapeDtypeStruct((N, D), jnp.float32),
        jax.ShapeDtypeStruct((N, 1), jnp.float32),
        jax.ShapeDtypeStruct((N, 1), jnp.float32),
    ],
)


def _tc2_body(pa, pb, t1, dinv, bg, wg2, t2_o):
    h = jnp.maximum((pa[...] + pb[...] + t1[...]) * dinv[...] + bg[...], 0.0)
    t2_o[...] = _dot(h, wg2[...]) * dinv[...]


_tc2 = pl.pallas_call(
    _tc2_body,
    grid=(GRID,),
    in_specs=[_row, _row, _row, _col, _b, _w],
    out_specs=_row,
    out_shape=jax.ShapeDtypeStruct((N, D), jnp.float32),
)


def _tc3_body(pa, pb, t2, dinv, bg, w_post, b_post, wfa, yd, bfin, y_o):
    h = jnp.maximum((pa[...] + pb[...] + t2[...]) * dinv[...] + bg[...], 0.0)
    feat = _dot(h, w_post[...]) + b_post[...]
    z = _dot(feat, wfa[...]) + yd[...] + bfin[...]
    y_o[...] = jax.nn.sigmoid(z)


_tc3 = pl.pallas_call(
    _tc3_body,
    grid=(GRID,),
    in_specs=[_row, _row, _row, _col, _b, _w, _b, _wcol, _col, _scal],
    out_specs=_col,
    out_shape=jax.ShapeDtypeStruct((N, 1), jnp.float32),
)


def kernel(x, edge_index, edge_attr, W_pre, b_pre, Wg1, bg1, Wg2, bg2,
           W_post, b_post, Wd0, bd0, Wd1, bd1, Wd2, bd2, Wd3, bd3,
           W_fin, b_fin):
    src = edge_index[0]
    dst3 = edge_index[1].reshape(NW, NCH, CHUNK)
    ones_c = jnp.ones((CHUNK,), jnp.float32)
    zeros1k = jnp.zeros((1000,), jnp.float32)
    zeros2d = jnp.zeros((ZR, D), jnp.float32)

    degs = _sc_degree(dst3, ones_c, zeros1k)
    d0 = degs[:N].reshape(N, 1)
    d1 = degs[N:].reshape(N, 1)

    ep = jnp.pad(edge_attr, ((0, 0), (0, D - KATTR)))
    wd0p = jnp.pad(Wd0, ((0, D - KATTR), (0, 0)))

    t1, dinv, yd = _tc1(
        x, d0, d1, ep, W_pre, b_pre.reshape(1, D), Wg1, wd0p,
        bd0.reshape(1, D), Wd1, bd1.reshape(1, D), Wd2, bd2.reshape(1, D),
        Wd3, bd3.reshape(1, D), W_fin[D:])

    p1 = _sc_aggregate(t1, src, dst3, zeros2d)
    t2 = _tc2(p1[:N], p1[N:], t1, dinv, bg1.reshape(1, D), Wg2)

    p2 = _sc_aggregate(t2, src, dst3, zeros2d)
    y = _tc3(p2[:N], p2[N:], t2, dinv, bg2.reshape(1, D), W_post,
             b_post.reshape(1, D), W_fin[:D], yd, b_fin.reshape(1, 1))
    return y.reshape(N)


# speedup vs baseline: 28.1292x; 1.0884x over previous
import functools

import jax
import jax.numpy as jnp
from jax import lax
from jax.experimental import pallas as pl
from jax.experimental.pallas import tpu as pltpu
from jax.experimental.pallas import tpu_sc as plsc

N = 10000
E = 320000
D = 128
KATTR = 5

NC = 2
NS = 16
NW = NC * NS
EPW = E // NW
CHUNK = 80
NCH = EPW // CHUNK
ZR = 80
NZCH = N // ZR

_mesh = plsc.VectorSubcoreMesh(core_axis_name="c", subcore_axis_name="s")


@functools.partial(
    pl.kernel,
    out_type=jax.ShapeDtypeStruct((2 * N,), jnp.float32),
    mesh=_mesh,
    scratch_types=[
        pltpu.VMEM((NCH, CHUNK), jnp.int32),
        pltpu.VMEM((CHUNK,), jnp.float32),
        pltpu.VMEM((1000,), jnp.float32),
        pltpu.VMEM_SHARED((N,), jnp.float32),
    ],
)
def _sc_degree(dst3, ones, zeros1k, out, didx, onesv, zbuf, accd):
    c = lax.axis_index("c")
    s = lax.axis_index("s")
    wid = c * NS + s
    pltpu.sync_copy(ones, onesv)
    pltpu.sync_copy(dst3.at[wid], didx)

    @pl.when(s < 10)
    def _zero():
        pltpu.sync_copy(zeros1k, zbuf)
        pltpu.sync_copy(zbuf, accd.at[pl.ds(s * 1000, 1000)])

    plsc.subcore_barrier()

    def body(i, carry):
        pltpu.sync_copy(onesv, accd.at[didx.at[i]], add=True)
        return carry

    lax.fori_loop(0, NCH, body, 0)
    plsc.subcore_barrier()

    @pl.when(s < 10)
    def _copy_out():
        pltpu.sync_copy(accd.at[pl.ds(s * 1000, 1000)], zbuf)
        pltpu.sync_copy(zbuf, out.at[pl.ds(c * N + s * 1000, 1000)])


NBUF = 3
PH_A = 64
PH_B = NCH - PH_A


@functools.partial(
    pl.kernel,
    out_type=jax.ShapeDtypeStruct((2 * N, D), jnp.float32),
    mesh=_mesh,
    scratch_types=[
        pltpu.VMEM((PH_A * CHUNK,), jnp.int32),
        pltpu.VMEM((PH_A, CHUNK), jnp.int32),
        [pltpu.VMEM((CHUNK, D), jnp.float32) for _ in range(NBUF)],
        pltpu.VMEM_SHARED((N, D), jnp.float32),
        [pltpu.SemaphoreType.DMA for _ in range(NBUF)],
        [pltpu.SemaphoreType.DMA for _ in range(NBUF)],
    ],
)
def _sc_aggregate(table, src, dst3, zeros, out, sidx, didx, rows, acc, gsem,
                  ssem):
    c = lax.axis_index("c")
    s = lax.axis_index("s")
    wid = c * NS + s
    base = wid * EPW
    pltpu.sync_copy(zeros, rows[0])
    for t in range((NZCH + NS - 1) // NS):
        ch = s + NS * t

        @pl.when(ch < NZCH)
        def _zero():
            pltpu.sync_copy(rows[0], acc.at[pl.ds(ch * ZR, ZR)])

    plsc.subcore_barrier()

    def drain(ref, sem):
        pltpu.make_async_copy(table.at[pl.ds(0, CHUNK)], ref, sem).wait()

    def gather(j, b):
        pltpu.async_copy(table.at[sidx.at[pl.ds(j * CHUNK, CHUNK)]], rows[b],
                         gsem[b])

    def phase(ch0, m):
        pltpu.sync_copy(src.at[pl.ds(base + ch0 * CHUNK, m * CHUNK)],
                        sidx.at[pl.ds(0, m * CHUNK)])
        pltpu.sync_copy(dst3.at[wid, pl.ds(ch0, m)], didx.at[pl.ds(0, m)])
        gather(0, 0)
        gather(1, 1)

        def body(k, carry):
            for b in range(NBUF):
                j = k * NBUF + b
                bp = (b + 2) % NBUF

                @pl.when(j < m)
                def _work():
                    drain(rows[b], gsem[b])
                    pltpu.async_copy(rows[b], acc.at[didx.at[j]], ssem[b],
                                    add=True)

                @pl.when(jnp.logical_and(j + 2 < m, j > 0))
                def _next():
                    drain(rows[bp], ssem[bp])
                    gather(j + 2, bp)

                @pl.when(j == 0)
                def _third():
                    gather(2, 2)

            return carry

        lax.fori_loop(0, (m + NBUF - 1) // NBUF, body, 0)
        for q in range(NBUF):
            drain(rows[q], ssem[(m - NBUF + q) % NBUF])

    phase(0, PH_A)
    phase(PH_A, PH_B)
    plsc.subcore_barrier()
    for t in range((NZCH + NS - 1) // NS):
        ch = s + NS * t

        @pl.when(ch < NZCH)
        def _copy_out():
            pltpu.sync_copy(acc.at[pl.ds(ch * ZR, ZR)], rows[0])
            pltpu.sync_copy(rows[0], out.at[pl.ds(c * N + ch * ZR, ZR)])


BLK = 400
GRID = N // BLK

_row = pl.BlockSpec((BLK, D), lambda i: (i, 0))
_col = pl.BlockSpec((BLK, 1), lambda i: (i, 0))
_w = pl.BlockSpec((D, D), lambda i: (0, 0))
_b = pl.BlockSpec((1, D), lambda i: (0, 0))
_wcol = pl.BlockSpec((D, 1), lambda i: (0, 0))
_scal = pl.BlockSpec((1, 1), lambda i: (0, 0))


def _dot(a, b):
    return jnp.dot(a, b, preferred_element_type=jnp.float32)


def _tc1_body(x, d0, d1, ep, w_pre, bp, wg1, wd0, bd0, wd1, bd1, wd2, bd2,
              wd3, bd3, wfb, t1_o, dinv_o, yd_o):
    dinv = lax.rsqrt(d0[...] + d1[...] + 1.0)
    dinv_o[...] = dinv
    h0 = _dot(x[...], w_pre[...]) + bp[...]
    t1_o[...] = _dot(h0, wg1[...]) * dinv
    e = jnp.maximum(_dot(ep[...], wd0[...]) + bd0[...], 0.0)
    e = jnp.maximum(_dot(e, wd1[...]) + bd1[...], 0.0)
    e = jnp.maximum(_dot(e, wd2[...]) + bd2[...], 0.0)
    dist = _dot(e, wd3[...]) + bd3[...]
    yd_o[...] = _dot(dist, wfb[...])


_tc1 = pl.pallas_call(
    _tc1_body,
    grid=(GRID,),
    in_specs=[_row, _col, _col, _row, _w, _b, _w, _w, _b, _w, _b, _w, _b,
              _w, _b, _wcol],
    out_specs=[_row, _col, _col],
    out_shape=[
        jax.ShapeDtypeStruct((N, D), jnp.float32),
        jax.ShapeDtypeStruct((N, 1), jnp.float32),
        jax.ShapeDtypeStruct((N, 1), jnp.float32),
    ],
)


def _tc2_body(pa, pb, t1, dinv, bg, wg2, t2_o):
    h = jnp.maximum((pa[...] + pb[...] + t1[...]) * dinv[...] + bg[...], 0.0)
    t2_o[...] = _dot(h, wg2[...]) * dinv[...]


_tc2 = pl.pallas_call(
    _tc2_body,
    grid=(GRID,),
    in_specs=[_row, _row, _row, _col, _b, _w],
    out_specs=_row,
    out_shape=jax.ShapeDtypeStruct((N, D), jnp.float32),
)


def _tc3_body(pa, pb, t2, dinv, bg, w_post, b_post, wfa, yd, bfin, y_o):
    h = jnp.maximum((pa[...] + pb[...] + t2[...]) * dinv[...] + bg[...], 0.0)
    feat = _dot(h, w_post[...]) + b_post[...]
    z = _dot(feat, wfa[...]) + yd[...] + bfin[...]
    y_o[...] = jax.nn.sigmoid(z)


_tc3 = pl.pallas_call(
    _tc3_body,
    grid=(GRID,),
    in_specs=[_row, _row, _row, _col, _b, _w, _b, _wcol, _col, _scal],
    out_specs=_col,
    out_shape=jax.ShapeDtypeStruct((N, 1), jnp.float32),
)


def kernel(x, edge_index, edge_attr, W_pre, b_pre, Wg1, bg1, Wg2, bg2,
           W_post, b_post, Wd0, bd0, Wd1, bd1, Wd2, bd2, Wd3, bd3,
           W_fin, b_fin):
    src = edge_index[0]
    dst3 = edge_index[1].reshape(NW, NCH, CHUNK)
    ones_c = jnp.ones((CHUNK,), jnp.float32)
    zeros1k = jnp.zeros((1000,), jnp.float32)
    zeros2d = jnp.zeros((ZR, D), jnp.float32)

    degs = _sc_degree(dst3, ones_c, zeros1k)
    d0 = degs[:N].reshape(N, 1)
    d1 = degs[N:].reshape(N, 1)

    ep = jnp.pad(edge_attr, ((0, 0), (0, D - KATTR)))
    wd0p = jnp.pad(Wd0, ((0, D - KATTR), (0, 0)))

    t1, dinv, yd = _tc1(
        x, d0, d1, ep, W_pre, b_pre.reshape(1, D), Wg1, wd0p,
        bd0.reshape(1, D), Wd1, bd1.reshape(1, D), Wd2, bd2.reshape(1, D),
        Wd3, bd3.reshape(1, D), W_fin[D:])

    p1 = _sc_aggregate(t1, src, dst3, zeros2d)
    t2 = _tc2(p1[:N], p1[N:], t1, dinv, bg1.reshape(1, D), Wg2)

    p2 = _sc_aggregate(t2, src, dst3, zeros2d)
    y = _tc3(p2[:N], p2[N:], t2, dinv, bg2.reshape(1, D), W_post,
             b_post.reshape(1, D), W_fin[:D], yd, b_fin.reshape(1, 1))
    return y.reshape(N)


# speedup vs baseline: 28.6149x; 1.0173x over previous
import functools

import jax
import jax.numpy as jnp
from jax import lax
from jax.experimental import pallas as pl
from jax.experimental.pallas import tpu as pltpu
from jax.experimental.pallas import tpu_sc as plsc

N = 10000
E = 320000
D = 128
KATTR = 5

NC = 2
NS = 16
NW = NC * NS
EPW = E // NW
CHUNK = 80
NCH = EPW // CHUNK
ZR = 80
NZCH = N // ZR

_mesh = plsc.VectorSubcoreMesh(core_axis_name="c", subcore_axis_name="s")


@functools.partial(
    pl.kernel,
    out_type=jax.ShapeDtypeStruct((2 * N,), jnp.float32),
    mesh=_mesh,
    scratch_types=[
        pltpu.VMEM((NCH, CHUNK), jnp.int32),
        pltpu.VMEM((CHUNK,), jnp.float32),
        pltpu.VMEM((1000,), jnp.float32),
        pltpu.VMEM_SHARED((N,), jnp.float32),
    ],
)
def _sc_degree(dst3, ones, zeros1k, out, didx, onesv, zbuf, accd):
    c = lax.axis_index("c")
    s = lax.axis_index("s")
    wid = c * NS + s
    pltpu.sync_copy(ones, onesv)
    pltpu.sync_copy(dst3.at[wid], didx)

    @pl.when(s < 10)
    def _zero():
        pltpu.sync_copy(zeros1k, zbuf)
        pltpu.sync_copy(zbuf, accd.at[pl.ds(s * 1000, 1000)])

    plsc.subcore_barrier()

    def body(i, carry):
        pltpu.sync_copy(onesv, accd.at[didx.at[i]], add=True)
        return carry

    lax.fori_loop(0, NCH, body, 0)
    plsc.subcore_barrier()

    @pl.when(s < 10)
    def _copy_out():
        pltpu.sync_copy(accd.at[pl.ds(s * 1000, 1000)], zbuf)
        pltpu.sync_copy(zbuf, out.at[pl.ds(c * N + s * 1000, 1000)])


NBUF = 3
PH_A = 64
PH_B = NCH - PH_A


@functools.partial(
    pl.kernel,
    out_type=jax.ShapeDtypeStruct((2 * N, D), jnp.float32),
    mesh=_mesh,
    scratch_types=[
        pltpu.VMEM((PH_A * CHUNK,), jnp.int32),
        pltpu.VMEM((PH_A, CHUNK), jnp.int32),
        [pltpu.VMEM((CHUNK, D), jnp.float32) for _ in range(NBUF)],
        pltpu.VMEM_SHARED((N, D), jnp.float32),
        [pltpu.SemaphoreType.DMA for _ in range(NBUF)],
        [pltpu.SemaphoreType.DMA for _ in range(NBUF)],
    ],
)
def _sc_aggregate(table, src, dst3, zeros, out, sidx, didx, rows, acc, gsem,
                  ssem):
    c = lax.axis_index("c")
    s = lax.axis_index("s")
    wid = c * NS + s
    base = wid * EPW
    pltpu.sync_copy(zeros, rows[0])
    for t in range((NZCH + NS - 1) // NS):
        ch = s + NS * t

        @pl.when(ch < NZCH)
        def _zero():
            pltpu.async_copy(rows[0], acc.at[pl.ds(ch * ZR, ZR)], gsem[0])

    for t in range((NZCH + NS - 1) // NS):
        ch = s + NS * t

        @pl.when(ch < NZCH)
        def _zwait():
            pltpu.make_async_copy(zeros, rows[1], gsem[0]).wait()

    plsc.subcore_barrier()

    def drain(ref, sem):
        pltpu.make_async_copy(table.at[pl.ds(0, CHUNK)], ref, sem).wait()

    def gather(j, b):
        pltpu.async_copy(table.at[sidx.at[pl.ds(j * CHUNK, CHUNK)]], rows[b],
                         gsem[b])

    def phase(ch0, m):
        pltpu.sync_copy(src.at[pl.ds(base + ch0 * CHUNK, m * CHUNK)],
                        sidx.at[pl.ds(0, m * CHUNK)])
        pltpu.sync_copy(dst3.at[wid, pl.ds(ch0, m)], didx.at[pl.ds(0, m)])
        gather(0, 0)
        gather(1, 1)

        def body(k, carry):
            for b in range(NBUF):
                j = k * NBUF + b
                bp = (b + 2) % NBUF

                @pl.when(j < m)
                def _work():
                    drain(rows[b], gsem[b])
                    pltpu.async_copy(rows[b], acc.at[didx.at[j]], ssem[b],
                                    add=True)

                @pl.when(jnp.logical_and(j + 2 < m, j > 0))
                def _next():
                    drain(rows[bp], ssem[bp])
                    gather(j + 2, bp)

                @pl.when(j == 0)
                def _third():
                    gather(2, 2)

            return carry

        lax.fori_loop(0, (m + NBUF - 1) // NBUF, body, 0)
        for q in range(NBUF):
            drain(rows[q], ssem[(m - NBUF + q) % NBUF])

    phase(0, PH_A)
    phase(PH_A, PH_B)
    plsc.subcore_barrier()
    for t in range((NZCH + NS - 1) // NS):
        ch = s + NS * t
        b = t % NBUF

        @pl.when(ch < NZCH)
        def _copy_out():
            if t >= NBUF:
                pltpu.make_async_copy(zeros, rows[b], ssem[b]).wait()
            pltpu.sync_copy(acc.at[pl.ds(ch * ZR, ZR)], rows[b])
            pltpu.async_copy(rows[b], out.at[pl.ds(c * N + ch * ZR, ZR)],
                             ssem[b])

    for b in range(NBUF):
        t_last = b

        @pl.when(s + NS * t_last < NZCH)
        def _cwait():
            pltpu.make_async_copy(zeros, rows[b], ssem[b]).wait()


BLK = 400
GRID = N // BLK

_row = pl.BlockSpec((BLK, D), lambda i: (i, 0))
_col = pl.BlockSpec((BLK, 1), lambda i: (i, 0))
_w = pl.BlockSpec((D, D), lambda i: (0, 0))
_b = pl.BlockSpec((1, D), lambda i: (0, 0))
_wcol = pl.BlockSpec((D, 1), lambda i: (0, 0))
_scal = pl.BlockSpec((1, 1), lambda i: (0, 0))


def _dot(a, b):
    return jnp.dot(a, b, preferred_element_type=jnp.float32)


def _tc1_body(x, d0, d1, ep, w_pre, bp, wg1, wd0, bd0, wd1, bd1, wd2, bd2,
              wd3, bd3, wfb, t1_o, dinv_o, yd_o):
    dinv = lax.rsqrt(d0[...] + d1[...] + 1.0)
    dinv_o[...] = dinv
    h0 = _dot(x[...], w_pre[...]) + bp[...]
    t1_o[...] = _dot(h0, wg1[...]) * dinv
    e = jnp.maximum(_dot(ep[...], wd0[...]) + bd0[...], 0.0)
    e = jnp.maximum(_dot(e, wd1[...]) + bd1[...], 0.0)
    e = jnp.maximum(_dot(e, wd2[...]) + bd2[...], 0.0)
    dist = _dot(e, wd3[...]) + bd3[...]
    yd_o[...] = _dot(dist, wfb[...])


_tc1 = pl.pallas_call(
    _tc1_body,
    grid=(GRID,),
    in_specs=[_row, _col, _col, pl.BlockSpec((BLK, KATTR), lambda i: (i, 0)),
              _w, _b, _w, pl.BlockSpec((KATTR, D), lambda i: (0, 0)), _b, _w,
              _b, _w, _b, _w, _b, _wcol],
    out_specs=[_row, _col, _col],
    out_shape=[
        jax.ShapeDtypeStruct((N, D), jnp.float32),
        jax.ShapeDtypeStruct((N, 1), jnp.float32),
        jax.ShapeDtypeStruct((N, 1), jnp.float32),
    ],
)


def _tc2_body(pa, pb, t1, dinv, bg, wg2, t2_o):
    h = jnp.maximum((pa[...] + pb[...] + t1[...]) * dinv[...] + bg[...], 0.0)
    t2_o[...] = _dot(h, wg2[...]) * dinv[...]


_tc2 = pl.pallas_call(
    _tc2_body,
    grid=(GRID,),
    in_specs=[_row, _row, _row, _col, _b, _w],
    out_specs=_row,
    out_shape=jax.ShapeDtypeStruct((N, D), jnp.float32),
)


def _tc3_body(pa, pb, t2, dinv, bg, w_post, b_post, wfa, yd, bfin, y_o):
    h = jnp.maximum((pa[...] + pb[...] + t2[...]) * dinv[...] + bg[...], 0.0)
    feat = _dot(h, w_post[...]) + b_post[...]
    z = _dot(feat, wfa[...]) + yd[...] + bfin[...]
    y_o[...] = jax.nn.sigmoid(z)


_tc3 = pl.pallas_call(
    _tc3_body,
    grid=(GRID,),
    in_specs=[_row, _row, _row, _col, _b, _w, _b, _wcol, _col, _scal],
    out_specs=_col,
    out_shape=jax.ShapeDtypeStruct((N, 1), jnp.float32),
)


def kernel(x, edge_index, edge_attr, W_pre, b_pre, Wg1, bg1, Wg2, bg2,
           W_post, b_post, Wd0, bd0, Wd1, bd1, Wd2, bd2, Wd3, bd3,
           W_fin, b_fin):
    src = edge_index[0]
    dst3 = edge_index[1].reshape(NW, NCH, CHUNK)
    ones_c = jnp.ones((CHUNK,), jnp.float32)
    zeros1k = jnp.zeros((1000,), jnp.float32)
    zeros2d = jnp.zeros((ZR, D), jnp.float32)

    degs = _sc_degree(dst3, ones_c, zeros1k)
    d0 = degs[:N].reshape(N, 1)
    d1 = degs[N:].reshape(N, 1)

    t1, dinv, yd = _tc1(
        x, d0, d1, edge_attr, W_pre, b_pre.reshape(1, D), Wg1, Wd0,
        bd0.reshape(1, D), Wd1, bd1.reshape(1, D), Wd2, bd2.reshape(1, D),
        Wd3, bd3.reshape(1, D), W_fin[D:])

    p1 = _sc_aggregate(t1, src, dst3, zeros2d)
    t2 = _tc2(p1[:N], p1[N:], t1, dinv, bg1.reshape(1, D), Wg2)

    p2 = _sc_aggregate(t2, src, dst3, zeros2d)
    y = _tc3(p2[:N], p2[N:], t2, dinv, bg2.reshape(1, D), W_post,
             b_post.reshape(1, D), W_fin[:D], yd, b_fin.reshape(1, 1))
    return y.reshape(N)
